# Initial kernel scaffold; baseline (speedup 1.0000x reference)
import jax
import jax.numpy as jnp
from jax.experimental import pallas as pl


def kernel(x, pos, edge_index, edge_attr, W_in, b_in, W_pre, b_pre, W_post, b_post, W_r1, b_r1, W_r2, b_r2):
    raise NotImplementedError("write your pallas kernel here")


# speedup vs baseline: 6.0155x; 6.0155x over previous
import functools

import jax
import jax.numpy as jnp
from jax import lax
from jax.experimental import pallas as pl
from jax.experimental.pallas import tpu as pltpu
from jax.experimental.pallas import tpu_sc as plsc

_N = 10000
_E = 320000
_D = 128
_DE = 16
_L = 5

_NC = 2
_NS = 16
_NW = _NC * _NS
_EPT = _E // _NW
_CH = 80
_NCHUNK = _EPT // _CH
_NP = 10240
_SPT = _NP // _NS
_LAST = _N - (_NS - 1) * _SPT

_mesh = plsc.VectorSubcoreMesh(core_axis_name="c", subcore_axis_name="s")


@functools.partial(
    pl.kernel,
    out_type=(jax.ShapeDtypeStruct((_NC, _N, _DE), jnp.float32),
              jax.ShapeDtypeStruct((_NC, _N, _DE), jnp.float32)),
    mesh=_mesh,
    scratch_types=[
        pltpu.VMEM((_CH,), jnp.int32),
        pltpu.VMEM((_CH,), jnp.int32),
        pltpu.VMEM((_CH, _DE), jnp.float32),
        pltpu.VMEM((_CH, _DE), jnp.float32),
        pltpu.VMEM_SHARED((_NP, _DE), jnp.float32),
        pltpu.VMEM_SHARED((_NP, _DE), jnp.float32),
        pltpu.SemaphoreType.DMA,
    ],
    compiler_params=pltpu.CompilerParams(use_tc_tiling_on_sc=False),
)
def _sc_precompute(p_hbm, ea_hbm, src_hbm, dst_hbm, z16_hbm,
                   outp_hbm, outea_hbm,
                   idx_s, idx_d, prow, earow, accp, accea, sem):
    c = lax.axis_index("c")
    s = lax.axis_index("s")
    w = c * _NS + s
    pltpu.sync_copy(z16_hbm, accp.at[pl.ds(s * _SPT, _SPT)])
    pltpu.sync_copy(z16_hbm, accea.at[pl.ds(s * _SPT, _SPT)])
    plsc.subcore_barrier()

    def body(i, carry):
        base = pl.multiple_of(w * _EPT + i * _CH, 8)
        pltpu.sync_copy(src_hbm.at[pl.ds(base, _CH)], idx_s)
        pltpu.sync_copy(dst_hbm.at[pl.ds(base, _CH)], idx_d)
        pltpu.async_copy(p_hbm.at[idx_s], prow, sem).wait()
        pltpu.sync_copy(ea_hbm.at[pl.ds(base, _CH)], earow)
        pltpu.sync_copy(prow, accp.at[idx_d], add=True)
        pltpu.sync_copy(earow, accea.at[idx_d], add=True)
        return carry

    lax.fori_loop(0, _NCHUNK, body, 0)
    plsc.subcore_barrier()

    @pl.when(s < _NS - 1)
    def _():
        pltpu.sync_copy(accp.at[pl.ds(s * _SPT, _SPT)],
                        outp_hbm.at[c, pl.ds(s * _SPT, _SPT)])
        pltpu.sync_copy(accea.at[pl.ds(s * _SPT, _SPT)],
                        outea_hbm.at[c, pl.ds(s * _SPT, _SPT)])

    @pl.when(s == _NS - 1)
    def _():
        pltpu.sync_copy(accp.at[pl.ds((_NS - 1) * _SPT, _LAST)],
                        outp_hbm.at[c, pl.ds((_NS - 1) * _SPT, _LAST)])
        pltpu.sync_copy(accea.at[pl.ds((_NS - 1) * _SPT, _LAST)],
                        outea_hbm.at[c, pl.ds((_NS - 1) * _SPT, _LAST)])


@functools.partial(
    pl.kernel,
    out_type=jax.ShapeDtypeStruct((_NC, _N, _D), jnp.float32),
    mesh=_mesh,
    scratch_types=[
        pltpu.VMEM((_CH,), jnp.int32),
        pltpu.VMEM((_CH,), jnp.int32),
        pltpu.VMEM((_CH, _D), jnp.float32),
        pltpu.VMEM_SHARED((_NP, _D), jnp.float32),
        pltpu.SemaphoreType.DMA,
    ],
)
def _sc_spmm(h_hbm, src_hbm, dst_hbm, z128_hbm, out_hbm,
             idx_s, idx_d, rows, acc, sem):
    c = lax.axis_index("c")
    s = lax.axis_index("s")
    w = c * _NS + s
    pltpu.sync_copy(z128_hbm, acc.at[pl.ds(s * _SPT, _SPT)])
    plsc.subcore_barrier()

    def body(i, carry):
        base = pl.multiple_of(w * _EPT + i * _CH, 8)
        pltpu.sync_copy(src_hbm.at[pl.ds(base, _CH)], idx_s)
        pltpu.sync_copy(dst_hbm.at[pl.ds(base, _CH)], idx_d)
        pltpu.async_copy(h_hbm.at[idx_s], rows, sem).wait()
        pltpu.sync_copy(rows, acc.at[idx_d], add=True)
        return carry

    lax.fori_loop(0, _NCHUNK, body, 0)
    plsc.subcore_barrier()

    @pl.when(s < _NS - 1)
    def _():
        pltpu.sync_copy(acc.at[pl.ds(s * _SPT, _SPT)],
                        out_hbm.at[c, pl.ds(s * _SPT, _SPT)])

    @pl.when(s == _NS - 1)
    def _():
        pltpu.sync_copy(acc.at[pl.ds((_NS - 1) * _SPT, _LAST)],
                        out_hbm.at[c, pl.ds((_NS - 1) * _SPT, _LAST)])


_R = 1000
_G = _N // _R


def _tc_pre_body(x_ref, pos_ref, w_ref, b_ref, h_ref, p_ref):
    x = x_ref[...]
    h_ref[...] = jnp.maximum(
        jnp.dot(x, w_ref[...], preferred_element_type=jnp.float32) + b_ref[...], 0.0)
    pos = pos_ref[...]
    q = jnp.sum(pos * pos, axis=1, keepdims=True)
    one = jnp.ones_like(q)
    zer = jnp.zeros((pos.shape[0], 6), jnp.float32)
    p_ref[...] = jnp.concatenate([pos, q, one, zer], axis=1)


def _tc_pre(x, pos_pad, w_in, b_in):
    return pl.pallas_call(
        _tc_pre_body,
        grid=(_G,),
        in_specs=[
            pl.BlockSpec((_R, _D), lambda i: (i, 0)),
            pl.BlockSpec((_R, 8), lambda i: (i, 0)),
            pl.BlockSpec((_D, _D), lambda i: (0, 0)),
            pl.BlockSpec((1, _D), lambda i: (0, 0)),
        ],
        out_specs=[
            pl.BlockSpec((_R, _D), lambda i: (i, 0)),
            pl.BlockSpec((_R, _DE), lambda i: (i, 0)),
        ],
        out_shape=[jax.ShapeDtypeStruct((_N, _D), jnp.float32),
                   jax.ShapeDtypeStruct((_N, _DE), jnp.float32)],
    )(x, pos_pad, w_in, b_in)


def _tc_layer_body(h_ref, sp_ref, ap_ref, aea_ref, p_ref,
                   wa_ref, wb_ref, we_ref, wd_ref, bpre_ref,
                   wp1_ref, wp2_ref, bpost_ref, out_ref):
    h = h_ref[...]
    s = sp_ref[0] + sp_ref[1]
    ap = ap_ref[0] + ap_ref[1]
    eas = aea_ref[0] + aea_ref[1]
    p = p_ref[...]
    q_sum = ap[:, 8:9]
    deg = ap[:, 9:10]
    q = p[:, 8:9]
    posdot = jnp.sum(p[:, :8] * ap[:, :8], axis=1, keepdims=True)
    sqd_sum = q_sum + deg * q - 2.0 * posdot
    m = (jnp.dot(s, wa_ref[...], preferred_element_type=jnp.float32)
         + jnp.dot(deg * h, wb_ref[...], preferred_element_type=jnp.float32)
         + jnp.dot(eas, we_ref[...], preferred_element_type=jnp.float32)
         + sqd_sum * wd_ref[...]
         + deg * bpre_ref[...])
    out_ref[...] = (jnp.dot(h, wp1_ref[...], preferred_element_type=jnp.float32)
                    + jnp.dot(m, wp2_ref[...], preferred_element_type=jnp.float32)
                    + bpost_ref[...] + h)


def _tc_layer(h, sp, accp, accea, p, wa, wb, we, wd, bpre, wp1, wp2, bpost):
    full = lambda r, c: pl.BlockSpec((r, c), lambda i: (0, 0))
    return pl.pallas_call(
        _tc_layer_body,
        grid=(_G,),
        in_specs=[
            pl.BlockSpec((_R, _D), lambda i: (i, 0)),
            pl.BlockSpec((_NC, _R, _D), lambda i: (0, i, 0)),
            pl.BlockSpec((_NC, _R, _DE), lambda i: (0, i, 0)),
            pl.BlockSpec((_NC, _R, _DE), lambda i: (0, i, 0)),
            pl.BlockSpec((_R, _DE), lambda i: (i, 0)),
            full(_D, _D), full(_D, _D), full(_DE, _D), full(1, _D), full(1, _D),
            full(_D, _D), full(_D, _D), full(1, _D),
        ],
        out_specs=pl.BlockSpec((_R, _D), lambda i: (i, 0)),
        out_shape=jax.ShapeDtypeStruct((_N, _D), jnp.float32),
    )(h, sp, accp, accea, p, wa, wb, we, wd, bpre, wp1, wp2, bpost)


def _tc_readout_body(h_ref, w1_ref, b1_ref, w2t_ref, b2_ref, out_ref):
    h = h_ref[...]
    mean = jnp.mean(h, axis=0, keepdims=True)
    mx = jnp.max(h, axis=0, keepdims=True)
    mm = jnp.concatenate([mean, mx], axis=1)
    z = jnp.maximum(
        jnp.dot(mm, w1_ref[...], preferred_element_type=jnp.float32) + b1_ref[...], 0.0)
    out_ref[...] = jnp.sum(z * w2t_ref[...], axis=1, keepdims=True) + b2_ref[...]


def _tc_readout(h, w1, b1, w2t, b2):
    return pl.pallas_call(
        _tc_readout_body,
        out_shape=jax.ShapeDtypeStruct((1, 1), jnp.float32),
    )(h, w1, b1, w2t, b2)


def kernel(x, pos, edge_index, edge_attr, W_in, b_in, W_pre, b_pre,
           W_post, b_post, W_r1, b_r1, W_r2, b_r2):
    src = edge_index[0]
    dst = edge_index[1]
    pos_pad = jnp.pad(pos, ((0, 0), (0, 5)))
    z16 = jnp.zeros((_SPT, _DE), jnp.float32)
    z128 = jnp.zeros((_SPT, _D), jnp.float32)

    h, p = _tc_pre(x, pos_pad, W_in, b_in.reshape(1, _D))
    accp, accea = _sc_precompute(p, edge_attr, src, dst, z16)

    for l in range(_L):
        sp = _sc_spmm(h, src, dst, z128)
        h = _tc_layer(
            h, sp, accp, accea, p,
            W_pre[l, :_D], W_pre[l, _D:2 * _D], W_pre[l, 2 * _D:2 * _D + _DE],
            W_pre[l, 2 * _D + _DE:], b_pre[l].reshape(1, _D),
            W_post[l, :_D], W_post[l, _D:], b_post[l].reshape(1, _D))

    return _tc_readout(h, W_r1, b_r1.reshape(1, _D),
                       W_r2.reshape(1, _D), b_r2.reshape(1, 1))


# speedup vs baseline: 10.8901x; 1.8103x over previous
import functools

import jax
import jax.numpy as jnp
from jax import lax
from jax.experimental import pallas as pl
from jax.experimental.pallas import tpu as pltpu
from jax.experimental.pallas import tpu_sc as plsc

_N = 10000
_E = 320000
_D = 128
_DE = 16
_L = 5

_NC = 2
_NS = 16
_NW = _NC * _NS
_EPT = _E // _NW
_CH = 80
_NCHUNK = _EPT // _CH
_NP = 10240
_SPT = _NP // _NS
_LAST = _N - (_NS - 1) * _SPT

_mesh = plsc.VectorSubcoreMesh(core_axis_name="c", subcore_axis_name="s")


@functools.partial(
    pl.kernel,
    out_type=(jax.ShapeDtypeStruct((_NC, _N, _DE), jnp.float32),
              jax.ShapeDtypeStruct((_NC, _N, _DE), jnp.float32)),
    mesh=_mesh,
    scratch_types=[
        pltpu.VMEM((_CH,), jnp.int32),
        pltpu.VMEM((_CH,), jnp.int32),
        pltpu.VMEM((_CH, _DE), jnp.float32),
        pltpu.VMEM((_CH, _DE), jnp.float32),
        pltpu.VMEM_SHARED((_NP, _DE), jnp.float32),
        pltpu.VMEM_SHARED((_NP, _DE), jnp.float32),
        pltpu.SemaphoreType.DMA,
    ],
    compiler_params=pltpu.CompilerParams(use_tc_tiling_on_sc=False),
)
def _sc_precompute(p_hbm, ea_hbm, src_hbm, dst_hbm, z16_hbm,
                   outp_hbm, outea_hbm,
                   idx_s, idx_d, prow, earow, accp, accea, sem):
    c = lax.axis_index("c")
    s = lax.axis_index("s")
    w = c * _NS + s
    pltpu.sync_copy(z16_hbm, accp.at[pl.ds(s * _SPT, _SPT)])
    pltpu.sync_copy(z16_hbm, accea.at[pl.ds(s * _SPT, _SPT)])
    plsc.subcore_barrier()

    def body(i, carry):
        base = pl.multiple_of(w * _EPT + i * _CH, 8)
        pltpu.sync_copy(src_hbm.at[pl.ds(base, _CH)], idx_s)
        pltpu.sync_copy(dst_hbm.at[pl.ds(base, _CH)], idx_d)
        pltpu.async_copy(p_hbm.at[idx_s], prow, sem).wait()
        pltpu.sync_copy(ea_hbm.at[pl.ds(base, _CH)], earow)
        pltpu.sync_copy(prow, accp.at[idx_d], add=True)
        pltpu.sync_copy(earow, accea.at[idx_d], add=True)
        return carry

    lax.fori_loop(0, _NCHUNK, body, 0)
    plsc.subcore_barrier()

    @pl.when(s < _NS - 1)
    def _():
        pltpu.sync_copy(accp.at[pl.ds(s * _SPT, _SPT)],
                        outp_hbm.at[c, pl.ds(s * _SPT, _SPT)])
        pltpu.sync_copy(accea.at[pl.ds(s * _SPT, _SPT)],
                        outea_hbm.at[c, pl.ds(s * _SPT, _SPT)])

    @pl.when(s == _NS - 1)
    def _():
        pltpu.sync_copy(accp.at[pl.ds((_NS - 1) * _SPT, _LAST)],
                        outp_hbm.at[c, pl.ds((_NS - 1) * _SPT, _LAST)])
        pltpu.sync_copy(accea.at[pl.ds((_NS - 1) * _SPT, _LAST)],
                        outea_hbm.at[c, pl.ds((_NS - 1) * _SPT, _LAST)])


@functools.partial(
    pl.kernel,
    out_type=jax.ShapeDtypeStruct((_NC, _N, _D), jnp.float32),
    mesh=_mesh,
    scratch_types=[
        pltpu.VMEM((_EPT,), jnp.int32),
        pltpu.VMEM((_EPT,), jnp.int32),
        pltpu.VMEM((_CH, _D), jnp.float32),
        pltpu.VMEM((_CH, _D), jnp.float32),
        pltpu.VMEM_SHARED((_NP, _D), jnp.float32),
        pltpu.SemaphoreType.DMA,
        pltpu.SemaphoreType.DMA,
    ],
)
def _sc_spmm(h_hbm, src_hbm, dst_hbm, z128_hbm, out_hbm,
             src_v, dst_v, rows_a, rows_b, acc, sem_a, sem_b):
    c = lax.axis_index("c")
    s = lax.axis_index("s")
    w = c * _NS + s
    ebase = pl.multiple_of(w * _EPT, 8)
    pltpu.sync_copy(src_hbm.at[pl.ds(ebase, _EPT)], src_v)
    pltpu.sync_copy(dst_hbm.at[pl.ds(ebase, _EPT)], dst_v)
    pltpu.sync_copy(z128_hbm, acc.at[pl.ds(s * _SPT, _SPT)])
    plsc.subcore_barrier()

    def gather(i, buf, sem):
        return pltpu.async_copy(
            h_hbm.at[src_v.at[pl.ds(i * _CH, _CH)]], buf, sem)

    def wait(buf, sem):
        pltpu.make_async_copy(
            h_hbm.at[src_v.at[pl.ds(0, _CH)]], buf, sem).wait()

    def scatter(i, buf):
        pltpu.sync_copy(buf, acc.at[dst_v.at[pl.ds(i * _CH, _CH)]], add=True)

    gather(0, rows_a, sem_a)

    def body(j, carry):
        i0 = j * 2
        gather(i0 + 1, rows_b, sem_b)
        wait(rows_a, sem_a)
        scatter(i0, rows_a)
        gather(i0 + 2, rows_a, sem_a)
        wait(rows_b, sem_b)
        scatter(i0 + 1, rows_b)
        return carry

    lax.fori_loop(0, (_NCHUNK - 1) // 2, body, 0)
    wait(rows_a, sem_a)
    scatter(_NCHUNK - 1, rows_a)
    plsc.subcore_barrier()

    @pl.when(s < _NS - 1)
    def _():
        pltpu.sync_copy(acc.at[pl.ds(s * _SPT, _SPT)],
                        out_hbm.at[c, pl.ds(s * _SPT, _SPT)])

    @pl.when(s == _NS - 1)
    def _():
        pltpu.sync_copy(acc.at[pl.ds((_NS - 1) * _SPT, _LAST)],
                        out_hbm.at[c, pl.ds((_NS - 1) * _SPT, _LAST)])


_R = 1000
_G = _N // _R


def _tc_pre_body(x_ref, pos_ref, w_ref, b_ref, h_ref, p_ref):
    x = x_ref[...]
    h_ref[...] = jnp.maximum(
        jnp.dot(x, w_ref[...], preferred_element_type=jnp.float32) + b_ref[...], 0.0)
    pos = pos_ref[...]
    q = jnp.sum(pos * pos, axis=1, keepdims=True)
    one = jnp.ones_like(q)
    zer = jnp.zeros((pos.shape[0], 6), jnp.float32)
    p_ref[...] = jnp.concatenate([pos, q, one, zer], axis=1)


def _tc_pre(x, pos_pad, w_in, b_in):
    return pl.pallas_call(
        _tc_pre_body,
        grid=(_G,),
        in_specs=[
            pl.BlockSpec((_R, _D), lambda i: (i, 0)),
            pl.BlockSpec((_R, 8), lambda i: (i, 0)),
            pl.BlockSpec((_D, _D), lambda i: (0, 0)),
            pl.BlockSpec((1, _D), lambda i: (0, 0)),
        ],
        out_specs=[
            pl.BlockSpec((_R, _D), lambda i: (i, 0)),
            pl.BlockSpec((_R, _DE), lambda i: (i, 0)),
        ],
        out_shape=[jax.ShapeDtypeStruct((_N, _D), jnp.float32),
                   jax.ShapeDtypeStruct((_N, _DE), jnp.float32)],
    )(x, pos_pad, w_in, b_in)


def _tc_layer_body(h_ref, sp_ref, ap_ref, aea_ref, p_ref,
                   wa_ref, wb_ref, we_ref, wd_ref, bpre_ref,
                   wp1_ref, wp2_ref, bpost_ref, out_ref):
    h = h_ref[...]
    s = sp_ref[0] + sp_ref[1]
    ap = ap_ref[0] + ap_ref[1]
    eas = aea_ref[0] + aea_ref[1]
    p = p_ref[...]
    q_sum = ap[:, 8:9]
    deg = ap[:, 9:10]
    q = p[:, 8:9]
    posdot = jnp.sum(p[:, :8] * ap[:, :8], axis=1, keepdims=True)
    sqd_sum = q_sum + deg * q - 2.0 * posdot
    m = (jnp.dot(s, wa_ref[...], preferred_element_type=jnp.float32)
         + jnp.dot(deg * h, wb_ref[...], preferred_element_type=jnp.float32)
         + jnp.dot(eas, we_ref[...], preferred_element_type=jnp.float32)
         + sqd_sum * wd_ref[...]
         + deg * bpre_ref[...])
    out_ref[...] = (jnp.dot(h, wp1_ref[...], preferred_element_type=jnp.float32)
                    + jnp.dot(m, wp2_ref[...], preferred_element_type=jnp.float32)
                    + bpost_ref[...] + h)


def _tc_layer(h, sp, accp, accea, p, wa, wb, we, wd, bpre, wp1, wp2, bpost):
    full = lambda r, c: pl.BlockSpec((r, c), lambda i: (0, 0))
    return pl.pallas_call(
        _tc_layer_body,
        grid=(_G,),
        in_specs=[
            pl.BlockSpec((_R, _D), lambda i: (i, 0)),
            pl.BlockSpec((_NC, _R, _D), lambda i: (0, i, 0)),
            pl.BlockSpec((_NC, _R, _DE), lambda i: (0, i, 0)),
            pl.BlockSpec((_NC, _R, _DE), lambda i: (0, i, 0)),
            pl.BlockSpec((_R, _DE), lambda i: (i, 0)),
            full(_D, _D), full(_D, _D), full(_DE, _D), full(1, _D), full(1, _D),
            full(_D, _D), full(_D, _D), full(1, _D),
        ],
        out_specs=pl.BlockSpec((_R, _D), lambda i: (i, 0)),
        out_shape=jax.ShapeDtypeStruct((_N, _D), jnp.float32),
    )(h, sp, accp, accea, p, wa, wb, we, wd, bpre, wp1, wp2, bpost)


def _tc_readout_body(h_ref, w1_ref, b1_ref, w2t_ref, b2_ref, out_ref):
    h = h_ref[...]
    mean = jnp.mean(h, axis=0, keepdims=True)
    mx = jnp.max(h, axis=0, keepdims=True)
    mm = jnp.concatenate([mean, mx], axis=1)
    z = jnp.maximum(
        jnp.dot(mm, w1_ref[...], preferred_element_type=jnp.float32) + b1_ref[...], 0.0)
    out_ref[...] = jnp.sum(z * w2t_ref[...], axis=1, keepdims=True) + b2_ref[...]


def _tc_readout(h, w1, b1, w2t, b2):
    return pl.pallas_call(
        _tc_readout_body,
        out_shape=jax.ShapeDtypeStruct((1, 1), jnp.float32),
    )(h, w1, b1, w2t, b2)


def kernel(x, pos, edge_index, edge_attr, W_in, b_in, W_pre, b_pre,
           W_post, b_post, W_r1, b_r1, W_r2, b_r2):
    src = edge_index[0]
    dst = edge_index[1]
    pos_pad = jnp.pad(pos, ((0, 0), (0, 5)))
    z16 = jnp.zeros((_SPT, _DE), jnp.float32)
    z128 = jnp.zeros((_SPT, _D), jnp.float32)

    h, p = _tc_pre(x, pos_pad, W_in, b_in.reshape(1, _D))
    accp, accea = _sc_precompute(p, edge_attr, src, dst, z16)

    for l in range(_L):
        sp = _sc_spmm(h, src, dst, z128)
        h = _tc_layer(
            h, sp, accp, accea, p,
            W_pre[l, :_D], W_pre[l, _D:2 * _D], W_pre[l, 2 * _D:2 * _D + _DE],
            W_pre[l, 2 * _D + _DE:], b_pre[l].reshape(1, _D),
            W_post[l, :_D], W_post[l, _D:], b_post[l].reshape(1, _D))

    return _tc_readout(h, W_r1, b_r1.reshape(1, _D),
                       W_r2.reshape(1, _D), b_r2.reshape(1, 1))


# speedup vs baseline: 13.5623x; 1.2454x over previous
import functools

import jax
import jax.numpy as jnp
from jax import lax
from jax.experimental import pallas as pl
from jax.experimental.pallas import tpu as pltpu
from jax.experimental.pallas import tpu_sc as plsc

_N = 10000
_E = 320000
_D = 128
_DE = 16
_L = 5

_NC = 2
_NS = 16
_NW = _NC * _NS
_EPT = _E // _NW
_CH = 80
_NCHUNK = _EPT // _CH
_NP = 10240
_SPT = _NP // _NS
_LAST = _N - (_NS - 1) * _SPT

_mesh = plsc.VectorSubcoreMesh(core_axis_name="c", subcore_axis_name="s")


@functools.partial(
    pl.kernel,
    out_type=(jax.ShapeDtypeStruct((_NC, _N, _DE), jnp.float32),
              jax.ShapeDtypeStruct((_NC, _N, _DE), jnp.float32)),
    mesh=_mesh,
    scratch_types=[
        pltpu.VMEM((_EPT,), jnp.int32),
        pltpu.VMEM((_EPT,), jnp.int32),
        pltpu.VMEM((_CH, _DE), jnp.float32),
        pltpu.VMEM((_CH, _DE), jnp.float32),
        pltpu.VMEM((_CH, _DE), jnp.float32),
        pltpu.VMEM((_CH, _DE), jnp.float32),
        pltpu.VMEM_SHARED((_NP, _DE), jnp.float32),
        pltpu.VMEM_SHARED((_NP, _DE), jnp.float32),
        pltpu.SemaphoreType.DMA,
        pltpu.SemaphoreType.DMA,
        pltpu.SemaphoreType.DMA,
        pltpu.SemaphoreType.DMA,
    ],
    compiler_params=pltpu.CompilerParams(use_tc_tiling_on_sc=False),
)
def _sc_precompute(p_hbm, ea_hbm, src_hbm, dst_hbm, z16_hbm,
                   outp_hbm, outea_hbm,
                   src_v, dst_v, prow_a, prow_b, ea_a, ea_b,
                   accp, accea, semp_a, semp_b, seme_a, seme_b):
    c = lax.axis_index("c")
    s = lax.axis_index("s")
    w = c * _NS + s
    ebase = pl.multiple_of(w * _EPT, 8)
    pltpu.sync_copy(src_hbm.at[pl.ds(ebase, _EPT)], src_v)
    pltpu.sync_copy(dst_hbm.at[pl.ds(ebase, _EPT)], dst_v)
    pltpu.sync_copy(z16_hbm, accp.at[pl.ds(s * _SPT, _SPT)])
    pltpu.sync_copy(z16_hbm, accea.at[pl.ds(s * _SPT, _SPT)])
    plsc.subcore_barrier()

    def fetch(i, pbuf, psem, ebuf, esem):
        pltpu.async_copy(p_hbm.at[src_v.at[pl.ds(i * _CH, _CH)]], pbuf, psem)
        base = pl.multiple_of(ebase + i * _CH, 8)
        pltpu.async_copy(ea_hbm.at[pl.ds(base, _CH)], ebuf, esem)

    def wait(pbuf, psem, ebuf, esem):
        pltpu.make_async_copy(
            p_hbm.at[src_v.at[pl.ds(0, _CH)]], pbuf, psem).wait()
        pltpu.make_async_copy(ea_hbm.at[pl.ds(0, _CH)], ebuf, esem).wait()

    def scatter(i, pbuf, ebuf):
        idx = dst_v.at[pl.ds(i * _CH, _CH)]
        pltpu.sync_copy(pbuf, accp.at[idx], add=True)
        pltpu.sync_copy(ebuf, accea.at[idx], add=True)

    fetch(0, prow_a, semp_a, ea_a, seme_a)

    def body(j, carry):
        i0 = j * 2
        fetch(i0 + 1, prow_b, semp_b, ea_b, seme_b)
        wait(prow_a, semp_a, ea_a, seme_a)
        scatter(i0, prow_a, ea_a)
        fetch(i0 + 2, prow_a, semp_a, ea_a, seme_a)
        wait(prow_b, semp_b, ea_b, seme_b)
        scatter(i0 + 1, prow_b, ea_b)
        return carry

    lax.fori_loop(0, (_NCHUNK - 1) // 2, body, 0)
    wait(prow_a, semp_a, ea_a, seme_a)
    scatter(_NCHUNK - 1, prow_a, ea_a)
    plsc.subcore_barrier()

    @pl.when(s < _NS - 1)
    def _():
        pltpu.sync_copy(accp.at[pl.ds(s * _SPT, _SPT)],
                        outp_hbm.at[c, pl.ds(s * _SPT, _SPT)])
        pltpu.sync_copy(accea.at[pl.ds(s * _SPT, _SPT)],
                        outea_hbm.at[c, pl.ds(s * _SPT, _SPT)])

    @pl.when(s == _NS - 1)
    def _():
        pltpu.sync_copy(accp.at[pl.ds((_NS - 1) * _SPT, _LAST)],
                        outp_hbm.at[c, pl.ds((_NS - 1) * _SPT, _LAST)])
        pltpu.sync_copy(accea.at[pl.ds((_NS - 1) * _SPT, _LAST)],
                        outea_hbm.at[c, pl.ds((_NS - 1) * _SPT, _LAST)])


@functools.partial(
    pl.kernel,
    out_type=jax.ShapeDtypeStruct((_NC, _N, _D), jnp.float32),
    mesh=_mesh,
    scratch_types=[
        pltpu.VMEM((_EPT,), jnp.int32),
        pltpu.VMEM((_EPT,), jnp.int32),
        pltpu.VMEM((_CH, _D), jnp.float32),
        pltpu.VMEM((_CH, _D), jnp.float32),
        pltpu.VMEM_SHARED((_NP, _D), jnp.float32),
        pltpu.SemaphoreType.DMA,
        pltpu.SemaphoreType.DMA,
    ],
)
def _sc_spmm(h_hbm, src_hbm, dst_hbm, z128_hbm, out_hbm,
             src_v, dst_v, rows_a, rows_b, acc, sem_a, sem_b):
    c = lax.axis_index("c")
    s = lax.axis_index("s")
    w = c * _NS + s
    ebase = pl.multiple_of(w * _EPT, 8)
    pltpu.sync_copy(src_hbm.at[pl.ds(ebase, _EPT)], src_v)
    pltpu.sync_copy(dst_hbm.at[pl.ds(ebase, _EPT)], dst_v)
    pltpu.sync_copy(z128_hbm, acc.at[pl.ds(s * _SPT, _SPT)])
    plsc.subcore_barrier()

    def gather(i, buf, sem):
        return pltpu.async_copy(
            h_hbm.at[src_v.at[pl.ds(i * _CH, _CH)]], buf, sem)

    def wait(buf, sem):
        pltpu.make_async_copy(
            h_hbm.at[src_v.at[pl.ds(0, _CH)]], buf, sem).wait()

    def scatter(i, buf):
        pltpu.sync_copy(buf, acc.at[dst_v.at[pl.ds(i * _CH, _CH)]], add=True)

    gather(0, rows_a, sem_a)

    def body(j, carry):
        i0 = j * 2
        gather(i0 + 1, rows_b, sem_b)
        wait(rows_a, sem_a)
        scatter(i0, rows_a)
        gather(i0 + 2, rows_a, sem_a)
        wait(rows_b, sem_b)
        scatter(i0 + 1, rows_b)
        return carry

    lax.fori_loop(0, (_NCHUNK - 1) // 2, body, 0)
    wait(rows_a, sem_a)
    scatter(_NCHUNK - 1, rows_a)
    plsc.subcore_barrier()

    @pl.when(s < _NS - 1)
    def _():
        pltpu.sync_copy(acc.at[pl.ds(s * _SPT, _SPT)],
                        out_hbm.at[c, pl.ds(s * _SPT, _SPT)])

    @pl.when(s == _NS - 1)
    def _():
        pltpu.sync_copy(acc.at[pl.ds((_NS - 1) * _SPT, _LAST)],
                        out_hbm.at[c, pl.ds((_NS - 1) * _SPT, _LAST)])


_R = 1000
_G = _N // _R


def _tc_pre_body(x_ref, pos_ref, w_ref, b_ref, h_ref, p_ref):
    x = x_ref[...]
    h_ref[...] = jnp.maximum(
        jnp.dot(x, w_ref[...], preferred_element_type=jnp.float32) + b_ref[...], 0.0)
    pos = pos_ref[...]
    q = jnp.sum(pos * pos, axis=1, keepdims=True)
    one = jnp.ones_like(q)
    zer = jnp.zeros((pos.shape[0], 6), jnp.float32)
    p_ref[...] = jnp.concatenate([pos, q, one, zer], axis=1)


def _tc_pre(x, pos_pad, w_in, b_in):
    return pl.pallas_call(
        _tc_pre_body,
        grid=(_G,),
        in_specs=[
            pl.BlockSpec((_R, _D), lambda i: (i, 0)),
            pl.BlockSpec((_R, 8), lambda i: (i, 0)),
            pl.BlockSpec((_D, _D), lambda i: (0, 0)),
            pl.BlockSpec((1, _D), lambda i: (0, 0)),
        ],
        out_specs=[
            pl.BlockSpec((_R, _D), lambda i: (i, 0)),
            pl.BlockSpec((_R, _DE), lambda i: (i, 0)),
        ],
        out_shape=[jax.ShapeDtypeStruct((_N, _D), jnp.float32),
                   jax.ShapeDtypeStruct((_N, _DE), jnp.float32)],
    )(x, pos_pad, w_in, b_in)


def _tc_layer_body(h_ref, sp_ref, ap_ref, aea_ref, p_ref,
                   wa_ref, wb_ref, we_ref, wd_ref, bpre_ref,
                   wp1_ref, wp2_ref, bpost_ref, out_ref):
    h = h_ref[...]
    s = sp_ref[0] + sp_ref[1]
    ap = ap_ref[0] + ap_ref[1]
    eas = aea_ref[0] + aea_ref[1]
    p = p_ref[...]
    q_sum = ap[:, 8:9]
    deg = ap[:, 9:10]
    q = p[:, 8:9]
    posdot = jnp.sum(p[:, :8] * ap[:, :8], axis=1, keepdims=True)
    sqd_sum = q_sum + deg * q - 2.0 * posdot
    m = (jnp.dot(s, wa_ref[...], preferred_element_type=jnp.float32)
         + jnp.dot(deg * h, wb_ref[...], preferred_element_type=jnp.float32)
         + jnp.dot(eas, we_ref[...], preferred_element_type=jnp.float32)
         + sqd_sum * wd_ref[...]
         + deg * bpre_ref[...])
    out_ref[...] = (jnp.dot(h, wp1_ref[...], preferred_element_type=jnp.float32)
                    + jnp.dot(m, wp2_ref[...], preferred_element_type=jnp.float32)
                    + bpost_ref[...] + h)


def _tc_layer(h, sp, accp, accea, p, wa, wb, we, wd, bpre, wp1, wp2, bpost):
    full = lambda r, c: pl.BlockSpec((r, c), lambda i: (0, 0))
    return pl.pallas_call(
        _tc_layer_body,
        grid=(_G,),
        in_specs=[
            pl.BlockSpec((_R, _D), lambda i: (i, 0)),
            pl.BlockSpec((_NC, _R, _D), lambda i: (0, i, 0)),
            pl.BlockSpec((_NC, _R, _DE), lambda i: (0, i, 0)),
            pl.BlockSpec((_NC, _R, _DE), lambda i: (0, i, 0)),
            pl.BlockSpec((_R, _DE), lambda i: (i, 0)),
            full(_D, _D), full(_D, _D), full(_DE, _D), full(1, _D), full(1, _D),
            full(_D, _D), full(_D, _D), full(1, _D),
        ],
        out_specs=pl.BlockSpec((_R, _D), lambda i: (i, 0)),
        out_shape=jax.ShapeDtypeStruct((_N, _D), jnp.float32),
    )(h, sp, accp, accea, p, wa, wb, we, wd, bpre, wp1, wp2, bpost)


def _tc_readout_body(h_ref, w1_ref, b1_ref, w2t_ref, b2_ref, out_ref):
    h = h_ref[...]
    mean = jnp.mean(h, axis=0, keepdims=True)
    mx = jnp.max(h, axis=0, keepdims=True)
    mm = jnp.concatenate([mean, mx], axis=1)
    z = jnp.maximum(
        jnp.dot(mm, w1_ref[...], preferred_element_type=jnp.float32) + b1_ref[...], 0.0)
    out_ref[...] = jnp.sum(z * w2t_ref[...], axis=1, keepdims=True) + b2_ref[...]


def _tc_readout(h, w1, b1, w2t, b2):
    return pl.pallas_call(
        _tc_readout_body,
        out_shape=jax.ShapeDtypeStruct((1, 1), jnp.float32),
    )(h, w1, b1, w2t, b2)


def kernel(x, pos, edge_index, edge_attr, W_in, b_in, W_pre, b_pre,
           W_post, b_post, W_r1, b_r1, W_r2, b_r2):
    src = edge_index[0]
    dst = edge_index[1]
    pos_pad = jnp.pad(pos, ((0, 0), (0, 5)))
    z16 = jnp.zeros((_SPT, _DE), jnp.float32)
    z128 = jnp.zeros((_SPT, _D), jnp.float32)

    h, p = _tc_pre(x, pos_pad, W_in, b_in.reshape(1, _D))
    accp, accea = _sc_precompute(p, edge_attr, src, dst, z16)

    for l in range(_L):
        sp = _sc_spmm(h, src, dst, z128)
        h = _tc_layer(
            h, sp, accp, accea, p,
            W_pre[l, :_D], W_pre[l, _D:2 * _D], W_pre[l, 2 * _D:2 * _D + _DE],
            W_pre[l, 2 * _D + _DE:], b_pre[l].reshape(1, _D),
            W_post[l, :_D], W_post[l, _D:], b_post[l].reshape(1, _D))

    return _tc_readout(h, W_r1, b_r1.reshape(1, _D),
                       W_r2.reshape(1, _D), b_r2.reshape(1, 1))
